# Initial kernel scaffold; baseline (speedup 1.0000x reference)
import functools

import jax
import jax.numpy as jnp
from jax import lax
from jax.experimental import pallas as pl
from jax.experimental.pallas import tpu as pltpu
from jax.experimental.pallas import tpu_sc as plsc

_B = 2
_C = 32
_N = 4096
_NCORE = 64
_OUTROW = _C * _NCORE
_NW = 32
_T = 16
_SLOTS = (_B * _N) // _NW
_BLOCKS = _SLOTS // _T


@functools.partial(
    pl.kernel,
    out_type=jax.ShapeDtypeStruct((_B * _N * _OUTROW,), jnp.float32),
    mesh=plsc.VectorSubcoreMesh(core_axis_name="c", subcore_axis_name="s"),
    scratch_types=[
        pltpu.VMEM((6, 6, _C, _T), jnp.float32),
        pltpu.VMEM((_T * _OUTROW,), jnp.float32),
        pltpu.SemaphoreType.DMA,
    ],
)
def _sc_transpose(x_hbm, out_hbm, in_b, out_b, sem):
    cid = lax.axis_index("c")
    sid = lax.axis_index("s")
    wid = sid * 2 + cid
    b = wid // 16
    nbase = (wid % 16) * _SLOTS

    stride = lax.iota(jnp.int32, _T) * _OUTROW

    zeros16 = jnp.zeros((_T,), jnp.float32)

    def zbody(i, carry):
        out_b[pl.ds(i * 16, 16)] = zeros16
        return carry

    lax.fori_loop(0, (_T * _OUTROW) // 16, zbody, 0)

    def block_body(k, carry):
        n0 = nbase + k * _T

        copies = []
        for g in range(6):
            src = x_hbm.at[pl.ds(64 * b + 8 * (g + 1) + 1, 6), :, pl.ds(n0, _T)]
            copies.append(pltpu.async_copy(src, in_b.at[g], sem))
        for cp in copies:
            cp.wait()

        def cbody(c, inner):
            c64 = c * _NCORE
            for g in range(6):
                for rj in range(6):
                    core = (g + 1) * 8 + (rj + 1)
                    vec = in_b[g, rj, c, :]
                    idx = stride + (c64 + core)
                    plsc.store_scatter(out_b, [idx], vec)
            return inner

        lax.fori_loop(0, _C, cbody, 0)

        ns0 = b * _N + n0
        pltpu.sync_copy(out_b, out_hbm.at[pl.ds(ns0 * _OUTROW, _T * _OUTROW)])
        return carry

    lax.fori_loop(0, _BLOCKS, block_body, 0)


def kernel(x):
    x3 = x.reshape(_B * _NCORE, _C, _N)
    out = _sc_transpose(x3)
    return out.reshape(_B, 16, 16, 16, _C, 8, 8)


# speedup vs baseline: 5.9919x; 5.9919x over previous
import functools

import jax
import jax.numpy as jnp
from jax import lax
from jax.experimental import pallas as pl
from jax.experimental.pallas import tpu as pltpu
from jax.experimental.pallas import tpu_sc as plsc

_B = 2
_C = 32
_N = 4096
_OUTROW = _C * 64
_HALF = 16
_HCOL = _HALF * 64
_NBLOCKS = (_B * _N) // 128
_BLK_PER_W = _NBLOCKS // 32


@functools.partial(
    pl.kernel,
    out_type=jax.ShapeDtypeStruct((_B * _N * _OUTROW,), jnp.float32),
    mesh=plsc.VectorSubcoreMesh(core_axis_name="c", subcore_axis_name="s"),
    compiler_params=pltpu.CompilerParams(needs_layout_passes=False),
    scratch_types=[
        pltpu.VMEM((6, 6, _HALF, 128), jnp.float32),
        pltpu.VMEM((16 * _HCOL,), jnp.float32),
        pltpu.SemaphoreType.DMA,
        pltpu.SemaphoreType.DMA,
    ],
)
def _sc_transpose(x_hbm, out_hbm, in_b, out_b, sem_in, sem_out):
    cid = lax.axis_index("c")
    sid = lax.axis_index("s")
    wid = sid * 2 + cid

    rows16 = lax.iota(jnp.int32, 16)
    rowmul = rows16 * _HCOL
    zeros16 = jnp.zeros((16,), jnp.float32)

    def zbody(i, carry):
        out_b[pl.ds(i * 16, 16)] = zeros16
        return carry

    lax.fori_loop(0, (16 * _HCOL) // 16, zbody, 0)

    def block_body(k, carry):
        blk = wid * _BLK_PER_W + k
        b = blk // 32
        nblk = blk % 32
        n0 = pl.multiple_of(nblk * 128, 128)
        base = b * _N + nblk * 128

        for h in range(2):
            c0 = h * _HALF
            pltpu.async_copy(
                x_hbm.at[b, pl.ds(1, 6), pl.ds(1, 6), pl.ds(c0, _HALF), pl.ds(n0, 128)],
                in_b,
                sem_in,
            ).wait()

            def s_body(s, cc):
                s16 = pl.multiple_of(s * 16, 16)

                def c_body(c, inner):
                    cbase = jnp.full((16,), c * 64, jnp.int32) + rowmul
                    for g in range(6):
                        for rj in range(6):
                            col = (g + 1) * 8 + (rj + 1)
                            vec = in_b[g, rj, c, pl.ds(s16, 16)]
                            plsc.store_scatter(out_b, [cbase + col], vec)
                    return inner

                lax.fori_loop(0, _HALF, c_body, 0)
                copies = []
                for t in range(16):
                    off = pl.multiple_of(
                        (base + s * 16 + t) * _OUTROW + h * _HCOL, _HCOL
                    )
                    copies.append(
                        pltpu.async_copy(
                            out_b.at[pl.ds(t * _HCOL, _HCOL)],
                            out_hbm.at[pl.ds(off, _HCOL)],
                            sem_out,
                        )
                    )
                for cp in copies:
                    cp.wait()
                return cc

            lax.fori_loop(0, 8, s_body, 0)
        return carry

    lax.fori_loop(0, _BLK_PER_W, block_body, 0)


def kernel(x):
    x5 = x.reshape(_B, 8, 8, _C, _N)
    out = _sc_transpose(x5)
    return out.reshape(_B, 16, 16, 16, _C, 8, 8)


# speedup vs baseline: 6.2788x; 1.0479x over previous
import functools

import jax
import jax.numpy as jnp
from jax import lax
from jax.experimental import pallas as pl
from jax.experimental.pallas import tpu as pltpu
from jax.experimental.pallas import tpu_sc as plsc

_B = 2
_C = 32
_N = 4096
_OUTROW = _C * 64
_CQ = 8
_QCOL = _CQ * 64
_NBLOCKS = (_B * _N) // 128
_BLK_PER_W = _NBLOCKS // 32


@functools.partial(
    pl.kernel,
    out_type=jax.ShapeDtypeStruct((_B * _N, _OUTROW), jnp.float32),
    mesh=plsc.VectorSubcoreMesh(core_axis_name="c", subcore_axis_name="s"),
    compiler_params=pltpu.CompilerParams(needs_layout_passes=False),
    scratch_types=[
        pltpu.VMEM((6, 6, _CQ, 128), jnp.float32),
        pltpu.VMEM((128, _QCOL), jnp.float32),
        pltpu.SemaphoreType.DMA,
        pltpu.SemaphoreType.DMA,
    ],
)
def _sc_transpose(x_hbm, out_hbm, in_b, out_b, sem_in, sem_out):
    cid = lax.axis_index("c")
    sid = lax.axis_index("s")
    wid = sid * 2 + cid

    rows16 = lax.iota(jnp.int32, 16)
    zeros16 = jnp.zeros((16,), jnp.float32)

    def zrow(r, carry):
        def zcol(j, inner):
            out_b[r, pl.ds(j * 16, 16)] = zeros16
            return inner

        return lax.fori_loop(0, _QCOL // 16, zcol, carry)

    lax.fori_loop(0, 128, zrow, 0)

    def block_body(k, carry):
        blk = wid * _BLK_PER_W + k
        b = blk // 32
        nblk = blk % 32
        n0 = pl.multiple_of(nblk * 128, 128)
        base = pl.multiple_of(b * _N + nblk * 128, 128)

        for q in range(4):
            c0 = q * _CQ
            pltpu.async_copy(
                x_hbm.at[b, pl.ds(1, 6), pl.ds(1, 6), pl.ds(c0, _CQ), pl.ds(n0, 128)],
                in_b,
                sem_in,
            ).wait()

            def s_body(s, cc):
                s16 = pl.multiple_of(s * 16, 16)
                rows = rows16 + s16

                def c_body(c, inner):
                    cbase = jnp.full((16,), c * 64, jnp.int32)
                    for g in range(6):
                        for rj in range(6):
                            col = (g + 1) * 8 + (rj + 1)
                            vec = in_b[g, rj, c, pl.ds(s16, 16)]
                            plsc.store_scatter(out_b, [rows, cbase + col], vec)
                    return inner

                return lax.fori_loop(0, _CQ, c_body, cc)

            lax.fori_loop(0, 8, s_body, 0)
            pltpu.sync_copy(
                out_b, out_hbm.at[pl.ds(base, 128), pl.ds(q * _QCOL, _QCOL)]
            )
        return carry

    lax.fori_loop(0, _BLK_PER_W, block_body, 0)


def kernel(x):
    x5 = x.reshape(_B, 8, 8, _C, _N)
    out = _sc_transpose(x5)
    return out.reshape(_B, 16, 16, 16, _C, 8, 8)
